# baseline (speedup 1.0000x reference)
import functools

import jax
import jax.numpy as jnp
from jax.experimental import pallas as pl
from jax.experimental.pallas import tpu as pltpu

TN = 1024


def _expert_body(x_ref, dp_ref, cb_ref, w1_ref, b1_ref, w2_ref, b2_ref,
                 o_ref, *, num_experts):
    e = pl.program_id(1)

    xt = x_ref[...]

    onehot = (jax.lax.broadcasted_iota(jnp.int32, (1, num_experts), 1)
              == e).astype(jnp.float32)
    disp = jnp.sum(dp_ref[:] * onehot, axis=1, keepdims=True)
    comb = jnp.sum(cb_ref[:] * onehot, axis=1, keepdims=True)

    h0 = jnp.dot(xt.astype(jnp.bfloat16), w1_ref[0].astype(jnp.bfloat16),
                 preferred_element_type=jnp.float32)

    xsum = jnp.sum(xt, axis=1, keepdims=True)
    mask = ((disp * xsum) != 0.0).astype(jnp.float32)

    h = h0 * disp + mask * b1_ref[0, 0][None, :]
    g = (0.5 * comb) * h * (1.0 + jax.lax.erf(h * 0.7071067811865476))

    y = jnp.dot(g.astype(jnp.bfloat16), w2_ref[0].astype(jnp.bfloat16),
                preferred_element_type=jnp.float32)

    @pl.when(e == 0)
    def _init():
        o_ref[...] = y + b2_ref[0][None, :]

    @pl.when(e > 0)
    def _accum():
        o_ref[...] += y


@jax.jit
def kernel(x, dispatch_tensor, combine_tensor, w1, b1, w2, b2):
    b, n, h = x.shape
    e, _, f = w1.shape
    tn = TN
    num_t = n // tn

    x2 = x.reshape(n, h)
    dp = dispatch_tensor.reshape(n, e)
    cb = combine_tensor.reshape(n, e)
    b1r = b1.reshape(e, 1, f)
    b2r = b2.reshape(1, h)

    out = pl.pallas_call(
        functools.partial(_expert_body, num_experts=e),
        grid=(num_t, e),
        in_specs=[
            pl.BlockSpec((tn, h), lambda ti, ei: (ti, 0)),
            pl.BlockSpec((tn, e), lambda ti, ei: (ti, 0)),
            pl.BlockSpec((tn, e), lambda ti, ei: (ti, 0)),
            pl.BlockSpec((1, h, f), lambda ti, ei: (ei, 0, 0)),
            pl.BlockSpec((1, 1, f), lambda ti, ei: (ei, 0, 0)),
            pl.BlockSpec((1, f, h), lambda ti, ei: (ei, 0, 0)),
            pl.BlockSpec((1, h), lambda ti, ei: (0, 0)),
        ],
        out_specs=pl.BlockSpec((tn, h), lambda ti, ei: (ti, 0)),
        out_shape=jax.ShapeDtypeStruct((n, h), jnp.float32),
        compiler_params=pltpu.CompilerParams(
            dimension_semantics=("arbitrary", "arbitrary"),
        ),
    )(x2, dp, cb, w1, b1r, w2, b2r)

    return out.reshape(b, n, h)


# speedup vs baseline: 1.0086x; 1.0086x over previous
import functools

import jax
import jax.numpy as jnp
from jax.experimental import pallas as pl
from jax.experimental.pallas import tpu as pltpu

TN = 1024


def _expert_body(x_ref, dp_ref, cb_ref, w1_ref, b1_ref, w2_ref, b2_ref,
                 o_ref, *, num_experts):
    e = pl.program_id(1)

    xt = x_ref[...]

    onehot = (jax.lax.broadcasted_iota(jnp.int32, (1, num_experts), 1)
              == e).astype(jnp.float32)
    disp = jnp.sum(dp_ref[:] * onehot, axis=1, keepdims=True)
    comb = jnp.sum(cb_ref[:] * onehot, axis=1, keepdims=True)

    h0 = jnp.dot(xt, w1_ref[0],
                 preferred_element_type=jnp.float32)

    xsum = jnp.sum(xt, axis=1, keepdims=True)
    mask = ((disp * xsum) != 0.0).astype(jnp.float32)

    h = h0 * disp + mask * b1_ref[0, 0][None, :]
    g = (0.5 * comb) * h * (1.0 + jax.lax.erf(h * 0.7071067811865476))

    y = jnp.dot(g, w2_ref[0],
                preferred_element_type=jnp.float32)

    @pl.when(e == 0)
    def _init():
        o_ref[...] = y + b2_ref[0][None, :]

    @pl.when(e > 0)
    def _accum():
        o_ref[...] += y


@jax.jit
def kernel(x, dispatch_tensor, combine_tensor, w1, b1, w2, b2):
    b, n, h = x.shape
    e, _, f = w1.shape
    tn = TN
    num_t = n // tn

    x2 = x.reshape(n, h)
    dp = dispatch_tensor.reshape(n, e)
    cb = combine_tensor.reshape(n, e)
    b1r = b1.reshape(e, 1, f)
    b2r = b2.reshape(1, h)

    out = pl.pallas_call(
        functools.partial(_expert_body, num_experts=e),
        grid=(num_t, e),
        in_specs=[
            pl.BlockSpec((tn, h), lambda ti, ei: (ti, 0)),
            pl.BlockSpec((tn, e), lambda ti, ei: (ti, 0)),
            pl.BlockSpec((tn, e), lambda ti, ei: (ti, 0)),
            pl.BlockSpec((1, h, f), lambda ti, ei: (ei, 0, 0)),
            pl.BlockSpec((1, 1, f), lambda ti, ei: (ei, 0, 0)),
            pl.BlockSpec((1, f, h), lambda ti, ei: (ei, 0, 0)),
            pl.BlockSpec((1, h), lambda ti, ei: (0, 0)),
        ],
        out_specs=pl.BlockSpec((tn, h), lambda ti, ei: (ti, 0)),
        out_shape=jax.ShapeDtypeStruct((n, h), jnp.float32),
        compiler_params=pltpu.CompilerParams(
            dimension_semantics=("parallel", "arbitrary"),
        ),
    )(x2, dp, cb, w1, b1r, w2, b2r)

    return out.reshape(b, n, h)
